# speedup vs baseline: 2.4839x; 2.4839x over previous
import functools

import jax
import jax.numpy as jnp
from jax import lax
from jax.experimental import pallas as pl
from jax.experimental.pallas import tpu as pltpu
from jax.experimental.pallas import tpu_sc as plsc


def kernel(h_idx, r_idx, t_idx, entity_emb, relation_emb):
    B = h_idx.shape[0]
    D = entity_emb.shape[1]
    info = plsc.get_sparse_core_info()
    NC, NS = info.num_cores, info.num_subcores
    NW = NC * NS
    b_per_w = B // NW

    mesh = plsc.VectorSubcoreMesh(core_axis_name="c", subcore_axis_name="s")
    out_sds = jax.ShapeDtypeStruct((B, D), jnp.float32)

    @functools.partial(
        pl.kernel,
        mesh=mesh,
        out_type=(out_sds, out_sds, out_sds),
        scratch_types=[
            pltpu.VMEM((b_per_w,), jnp.int32),
            pltpu.VMEM((b_per_w, D), jnp.float32),
            pltpu.SemaphoreType.DMA,
        ],
    )
    def k(h_hbm, r_hbm, t_hbm, ent_hbm, rel_hbm, h_out, r_out, t_out,
          idx_v, rows_v, sem):
        wid = lax.axis_index("s") * NC + lax.axis_index("c")
        base = wid * b_per_w

        pltpu.sync_copy(h_hbm.at[pl.ds(base, b_per_w)], idx_v)
        pltpu.async_copy(ent_hbm.at[idx_v], rows_v, sem).wait()
        pltpu.sync_copy(rows_v, h_out.at[pl.ds(base, b_per_w)])

        pltpu.sync_copy(r_hbm.at[pl.ds(base, b_per_w)], idx_v)
        pltpu.async_copy(rel_hbm.at[idx_v], rows_v, sem).wait()
        pltpu.sync_copy(rows_v, r_out.at[pl.ds(base, b_per_w)])

        pltpu.sync_copy(t_hbm.at[pl.ds(base, b_per_w)], idx_v)
        pltpu.async_copy(ent_hbm.at[idx_v], rows_v, sem).wait()
        pltpu.sync_copy(rows_v, t_out.at[pl.ds(base, b_per_w)])

    return k(h_idx, r_idx, t_idx, entity_emb, relation_emb)


# speedup vs baseline: 2.4930x; 1.0037x over previous
import functools

import jax
import jax.numpy as jnp
from jax import lax
from jax.experimental import pallas as pl
from jax.experimental.pallas import tpu as pltpu
from jax.experimental.pallas import tpu_sc as plsc

_CHUNK = 128
_NBUF = 4
_INFLIGHT = 3


def kernel(h_idx, r_idx, t_idx, entity_emb, relation_emb):
    B = h_idx.shape[0]
    D = entity_emb.shape[1]
    info = plsc.get_sparse_core_info()
    NC, NS = info.num_cores, info.num_subcores
    NW = NC * NS
    b_per_w = B // NW
    n_chunks = b_per_w // _CHUNK

    mesh = plsc.VectorSubcoreMesh(core_axis_name="c", subcore_axis_name="s")
    out_sds = jax.ShapeDtypeStruct((B, D), jnp.float32)

    @functools.partial(
        pl.kernel,
        mesh=mesh,
        out_type=(out_sds, out_sds, out_sds),
        scratch_types=(
            [pltpu.VMEM((3 * b_per_w,), jnp.int32)]
            + [pltpu.VMEM((_CHUNK, D), jnp.float32) for _ in range(_NBUF)]
            + [pltpu.SemaphoreType.DMA for _ in range(2 * _NBUF)]
        ),
    )
    def k(h_hbm, r_hbm, t_hbm, ent_hbm, rel_hbm, h_out, r_out, t_out,
          idx_v, *bufs_and_sems):
        bufs = bufs_and_sems[:_NBUF]
        gsem = bufs_and_sems[_NBUF:2 * _NBUF]
        ssem = bufs_and_sems[2 * _NBUF:]

        wid = lax.axis_index("s") * NC + lax.axis_index("c")
        base = wid * b_per_w

        pltpu.sync_copy(h_hbm.at[pl.ds(base, b_per_w)],
                        idx_v.at[pl.ds(0, b_per_w)])
        pltpu.sync_copy(r_hbm.at[pl.ds(base, b_per_w)],
                        idx_v.at[pl.ds(b_per_w, b_per_w)])
        pltpu.sync_copy(t_hbm.at[pl.ds(base, b_per_w)],
                        idx_v.at[pl.ds(2 * b_per_w, b_per_w)])

        tasks = []
        for t, (table, out) in enumerate(
                ((ent_hbm, h_out), (rel_hbm, r_out), (ent_hbm, t_out))):
            for c in range(n_chunks):
                tasks.append((table, t * b_per_w + c * _CHUNK, out,
                              base + c * _CHUNK))
        T = len(tasks)

        gathers = [None] * T
        stores = [None] * T

        def gather_start(s):
            table, ioff, _, _ = tasks[s]
            b = s % _NBUF
            gathers[s] = pltpu.async_copy(
                table.at[idx_v.at[pl.ds(ioff, _CHUNK)]], bufs[b], gsem[b])

        def store_start(s):
            _, _, out, obase = tasks[s]
            b = s % _NBUF
            stores[s] = pltpu.async_copy(
                bufs[b], out.at[pl.ds(obase, _CHUNK)], ssem[b])

        for s in range(min(_INFLIGHT, T)):
            gather_start(s)
        for s in range(T):
            if s >= 1:
                stores[s - 1].wait()
            if s + _INFLIGHT < T:
                gather_start(s + _INFLIGHT)
            gathers[s].wait()
            store_start(s)
        stores[T - 1].wait()

    return k(h_idx, r_idx, t_idx, entity_emb, relation_emb)
